# Initial kernel scaffold; baseline (speedup 1.0000x reference)
import jax
import jax.numpy as jnp
from jax.experimental import pallas as pl


def kernel(z, objs, triples, attributes, params):
    raise NotImplementedError("write your pallas kernel here")


# speedup vs baseline: 2.3504x; 2.3504x over previous
import functools

import jax
import jax.numpy as jnp
from jax import lax
from jax.experimental import pallas as pl
from jax.experimental.pallas import tpu as pltpu
from jax.experimental.pallas import tpu_sc as plsc

F32 = jnp.float32
I32 = jnp.int32

N_OBJS = 36
N_PREDS = 16
N_ATTRS = 8
EMB = 128
H = 512
N = 10000
E = 160000
N2 = 10240
E2 = 163840

NC = 2
NS = 16
NW = NC * NS

EB = 1280
NB = 1280
GCH = E2 // NW
GIT = GCH // 128
SCH = E2 // NS
SIT = SCH // 128
NROWS = N2 // NS

_mesh = plsc.VectorSubcoreMesh(core_axis_name="c", subcore_axis_name="s")


@functools.partial(
    pl.kernel,
    mesh=_mesh,
    out_type=[jax.ShapeDtypeStruct((E2, EMB), F32),
              jax.ShapeDtypeStruct((E2, EMB), F32)],
    scratch_types=[pltpu.VMEM((GIT, 128), I32),
                   pltpu.VMEM((GIT, 128), I32),
                   pltpu.VMEM((128, EMB), F32),
                   pltpu.VMEM((128, EMB), F32),
                   pltpu.SemaphoreType.DMA,
                   pltpu.SemaphoreType.DMA],
)
def _sc_gather(table_hbm, sto_hbm, cs_hbm, co_hbm,
               sidx_v, oidx_v, srows_v, orows_v, sem1, sem2):
    cid = lax.axis_index("c")
    sid = lax.axis_index("s")
    wid = sid * NC + cid
    pltpu.sync_copy(sto_hbm.at[0, pl.ds(wid * GIT, GIT)], sidx_v)
    pltpu.sync_copy(sto_hbm.at[1, pl.ds(wid * GIT, GIT)], oidx_v)

    def body(j, carry):
        base = wid * GCH + j * 128
        pltpu.async_copy(table_hbm.at[sidx_v.at[j]], srows_v, sem1).wait()
        pltpu.sync_copy(srows_v, cs_hbm.at[pl.ds(base, 128)])
        pltpu.async_copy(table_hbm.at[oidx_v.at[j]], orows_v, sem2).wait()
        pltpu.sync_copy(orows_v, co_hbm.at[pl.ds(base, 128)])
        return carry

    lax.fori_loop(0, GIT, body, 0)


@functools.partial(
    pl.kernel,
    mesh=_mesh,
    out_type=jax.ShapeDtypeStruct((4, N2, 128), F32),
    scratch_types=[pltpu.VMEM((SIT, 128), I32),
                   pltpu.VMEM((SIT, 128), I32),
                   pltpu.VMEM((128, 128), F32),
                   pltpu.VMEM_SHARED((N2, 128), F32)],
)
def _sc_scatter(news_hbm, newo_hbm, sto_hbm, zeros_hbm, out_hbm,
                sidx_v, oidx_v, blk_v, pool_sp):
    cid = lax.axis_index("c")
    sid = lax.axis_index("s")
    pltpu.sync_copy(sto_hbm.at[0, pl.ds(sid * SIT, SIT)], sidx_v)
    pltpu.sync_copy(sto_hbm.at[1, pl.ds(sid * SIT, SIT)], oidx_v)
    for chunk in range(2):
        c = cid * 2 + chunk
        pltpu.sync_copy(zeros_hbm, pool_sp.at[pl.ds(sid * NROWS, NROWS)])
        plsc.subcore_barrier()

        def body(j, carry):
            base = sid * SCH + j * 128
            pltpu.sync_copy(news_hbm.at[c, pl.ds(base, 128)], blk_v)
            pltpu.sync_copy(blk_v, pool_sp.at[sidx_v.at[j]], add=True)
            pltpu.sync_copy(newo_hbm.at[c, pl.ds(base, 128)], blk_v)
            pltpu.sync_copy(blk_v, pool_sp.at[oidx_v.at[j]], add=True)
            return carry

        lax.fori_loop(0, SIT, body, 0)
        plsc.subcore_barrier()
        pltpu.sync_copy(pool_sp.at[pl.ds(sid * NROWS, NROWS)],
                        out_hbm.at[c, pl.ds(sid * NROWS, NROWS)])
        plsc.subcore_barrier()


@functools.partial(
    pl.kernel,
    mesh=_mesh,
    out_type=jax.ShapeDtypeStruct((2, N2, 16), F32),
    scratch_types=[pltpu.VMEM((SIT, 128), I32),
                   pltpu.VMEM((128, 16), F32),
                   pltpu.VMEM_SHARED((N2, 16), F32)],
)
def _sc_counts(sto_hbm, ones_hbm, zeros_hbm, out_hbm, idx_v, ones_v, cnt_sp):
    cid = lax.axis_index("c")
    sid = lax.axis_index("s")
    pltpu.sync_copy(sto_hbm.at[cid, pl.ds(sid * SIT, SIT)], idx_v)
    pltpu.sync_copy(ones_hbm, ones_v)
    pltpu.sync_copy(zeros_hbm, cnt_sp.at[pl.ds(sid * NROWS, NROWS)])
    plsc.subcore_barrier()

    def body(j, carry):
        pltpu.sync_copy(ones_v, cnt_sp.at[idx_v.at[j]], add=True)
        return carry

    lax.fori_loop(0, SIT, body, 0)
    plsc.subcore_barrier()
    pltpu.sync_copy(cnt_sp.at[pl.ds(sid * NROWS, NROWS)],
                    out_hbm.at[cid, pl.ds(sid * NROWS, NROWS)])


def _embed_body(objs_ref, attr_ref, oemb_ref, aemb_ref, out_ref):
    oids = objs_ref[...]
    aids = attr_ref[...]
    oh_o = (oids == lax.broadcasted_iota(I32, (NB, N_OBJS), 1)).astype(F32)
    oh_a = (aids == lax.broadcasted_iota(I32, (NB, N_ATTRS), 1)).astype(F32)
    ov = jnp.dot(oh_o, oemb_ref[...], preferred_element_type=F32)
    av = jnp.dot(oh_a, aemb_ref[...], preferred_element_type=F32)
    out_ref[...] = jnp.concatenate([ov, av], axis=1)


def _embed(objs2d, attr2d, oemb, aemb):
    return pl.pallas_call(
        _embed_body,
        grid=(N2 // NB,),
        in_specs=[pl.BlockSpec((NB, 1), lambda i: (i, 0)),
                  pl.BlockSpec((NB, 1), lambda i: (i, 0)),
                  pl.BlockSpec(oemb.shape, lambda i: (0, 0)),
                  pl.BlockSpec(aemb.shape, lambda i: (0, 0))],
        out_specs=pl.BlockSpec((NB, EMB), lambda i: (i, 0)),
        out_shape=jax.ShapeDtypeStruct((N2, EMB), F32),
    )(objs2d, attr2d, oemb, aemb)


def _pred_body(p_ref, pemb_ref, out_ref):
    pids = p_ref[...]
    oh = (pids == lax.broadcasted_iota(I32, (EB, N_PREDS), 1)).astype(F32)
    out_ref[...] = jnp.dot(oh, pemb_ref[...], preferred_element_type=F32)


def _pred_embed(p2d, pemb):
    return pl.pallas_call(
        _pred_body,
        grid=(E2 // EB,),
        in_specs=[pl.BlockSpec((EB, 1), lambda i: (i, 0)),
                  pl.BlockSpec(pemb.shape, lambda i: (0, 0))],
        out_specs=pl.BlockSpec((EB, EMB), lambda i: (i, 0)),
        out_shape=jax.ShapeDtypeStruct((E2, EMB), F32),
    )(p2d, pemb)


def _edge_body(cs_ref, pv_ref, co_ref, w1_ref, b1_ref, w2_ref, b2_ref,
               ns_ref, np_ref, no_ref):
    h = (jnp.dot(cs_ref[...], w1_ref[0:EMB], preferred_element_type=F32)
         + jnp.dot(pv_ref[...], w1_ref[EMB:2 * EMB], preferred_element_type=F32)
         + jnp.dot(co_ref[...], w1_ref[2 * EMB:3 * EMB], preferred_element_type=F32)
         + b1_ref[...])
    h = jnp.maximum(h, 0.0)
    for c in range(4):
        ns_ref[c] = jnp.maximum(
            jnp.dot(h, w2_ref[:, c * 128:(c + 1) * 128],
                    preferred_element_type=F32) + b2_ref[:, c * 128:(c + 1) * 128],
            0.0)
    np_ref[...] = jnp.maximum(
        jnp.dot(h, w2_ref[:, 512:640], preferred_element_type=F32)
        + b2_ref[:, 512:640], 0.0)
    for c in range(4):
        lo = 640 + c * 128
        no_ref[c] = jnp.maximum(
            jnp.dot(h, w2_ref[:, lo:lo + 128], preferred_element_type=F32)
            + b2_ref[:, lo:lo + 128], 0.0)


def _edge_mlp(cs, pv, co, w1, b1, w2, b2):
    return pl.pallas_call(
        _edge_body,
        grid=(E2 // EB,),
        in_specs=[pl.BlockSpec((EB, EMB), lambda i: (i, 0)),
                  pl.BlockSpec((EB, EMB), lambda i: (i, 0)),
                  pl.BlockSpec((EB, EMB), lambda i: (i, 0)),
                  pl.BlockSpec(w1.shape, lambda i: (0, 0)),
                  pl.BlockSpec(b1.shape, lambda i: (0, 0)),
                  pl.BlockSpec(w2.shape, lambda i: (0, 0)),
                  pl.BlockSpec(b2.shape, lambda i: (0, 0))],
        out_specs=[pl.BlockSpec((4, EB, 128), lambda i: (0, i, 0)),
                   pl.BlockSpec((EB, EMB), lambda i: (i, 0)),
                   pl.BlockSpec((4, EB, 128), lambda i: (0, i, 0))],
        out_shape=[jax.ShapeDtypeStruct((4, E2, 128), F32),
                   jax.ShapeDtypeStruct((E2, EMB), F32),
                   jax.ShapeDtypeStruct((4, E2, 128), F32)],
    )(cs, pv, co, w1, b1, w2, b2)


def _node_body(p4_ref, cnt_ref, w1_ref, b1_ref, w2_ref, b2_ref, out_ref):
    cnt = cnt_ref[0][:, 0:1] + cnt_ref[1][:, 0:1]
    inv = 1.0 / jnp.maximum(cnt, 1.0)
    x = jnp.concatenate([p4_ref[0], p4_ref[1], p4_ref[2], p4_ref[3]], axis=1)
    x = x * inv
    h = jnp.maximum(jnp.dot(x, w1_ref[...], preferred_element_type=F32)
                    + b1_ref[...], 0.0)
    out_ref[...] = jnp.maximum(
        jnp.dot(h, w2_ref[...], preferred_element_type=F32) + b2_ref[...], 0.0)


def _node_mlp(p4, cnts, w1, b1, w2, b2):
    return pl.pallas_call(
        _node_body,
        grid=(N2 // NB,),
        in_specs=[pl.BlockSpec((4, NB, 128), lambda i: (0, i, 0)),
                  pl.BlockSpec((2, NB, 16), lambda i: (0, i, 0)),
                  pl.BlockSpec(w1.shape, lambda i: (0, 0)),
                  pl.BlockSpec(b1.shape, lambda i: (0, 0)),
                  pl.BlockSpec(w2.shape, lambda i: (0, 0)),
                  pl.BlockSpec(b2.shape, lambda i: (0, 0))],
        out_specs=pl.BlockSpec((NB, EMB), lambda i: (i, 0)),
        out_shape=jax.ShapeDtypeStruct((N2, EMB), F32),
    )(p4, cnts, w1, b1, w2, b2)


HB = 1000


def _heads_body(obj_ref, obj0_ref, z_ref,
                wb1_ref, bb1_ref, wb2_ref, bb2_ref,
                wa1_ref, ba1_ref, wa2_ref, ba2_ref,
                box_ref, ang_ref):
    obj = obj_ref[...]
    zz = z_ref[...]
    attr = obj0_ref[:, 96:128]
    bx = jnp.concatenate([obj, zz, attr], axis=1)
    hb = jnp.maximum(jnp.dot(bx, wb1_ref[...], preferred_element_type=F32)
                     + bb1_ref[...], 0.0)
    box_ref[...] = jnp.dot(hb, wb2_ref[...], preferred_element_type=F32) + bb2_ref[...]
    ax = jnp.concatenate([obj, zz], axis=1)
    ha = jnp.maximum(jnp.dot(ax, wa1_ref[...], preferred_element_type=F32)
                     + ba1_ref[...], 0.0)
    a = jnp.dot(ha, wa2_ref[...], preferred_element_type=F32) + ba2_ref[...]
    m = jnp.max(a, axis=1, keepdims=True)
    lse = m + jnp.log(jnp.sum(jnp.exp(a - m), axis=1, keepdims=True))
    ang_ref[...] = a - lse


def _heads(obj, obj0, z, wb1, bb1, wb2, bb2, wa1, ba1, wa2, ba2):
    return pl.pallas_call(
        _heads_body,
        grid=(N // HB,),
        in_specs=[pl.BlockSpec((HB, EMB), lambda i: (i, 0)),
                  pl.BlockSpec((HB, EMB), lambda i: (i, 0)),
                  pl.BlockSpec((HB, EMB), lambda i: (i, 0)),
                  pl.BlockSpec(wb1.shape, lambda i: (0, 0)),
                  pl.BlockSpec(bb1.shape, lambda i: (0, 0)),
                  pl.BlockSpec(wb2.shape, lambda i: (0, 0)),
                  pl.BlockSpec(bb2.shape, lambda i: (0, 0)),
                  pl.BlockSpec(wa1.shape, lambda i: (0, 0)),
                  pl.BlockSpec(ba1.shape, lambda i: (0, 0)),
                  pl.BlockSpec(wa2.shape, lambda i: (0, 0)),
                  pl.BlockSpec(ba2.shape, lambda i: (0, 0))],
        out_specs=[pl.BlockSpec((HB, 128), lambda i: (i, 0)),
                   pl.BlockSpec((HB, 128), lambda i: (i, 0))],
        out_shape=[jax.ShapeDtypeStruct((N, 128), F32),
                   jax.ShapeDtypeStruct((N, 128), F32)],
    )(obj, obj0, z, wb1, bb1, wb2, bb2, wa1, ba1, wa2, ba2)


def kernel(z, objs, triples, attributes, params):
    s = triples[:, 0]
    p = triples[:, 1]
    o = triples[:, 2]
    pad_e = E2 - E
    sp = jnp.concatenate([s, jnp.full((pad_e,), N, I32)])
    op = jnp.concatenate([o, jnp.full((pad_e,), N, I32)])
    pp = jnp.concatenate([p, jnp.zeros((pad_e,), I32)])
    sto = jnp.stack([sp.reshape(E2 // 128, 128), op.reshape(E2 // 128, 128)])

    pad_n = N2 - N
    objs2d = jnp.concatenate([objs, jnp.full((pad_n,), N_OBJS, I32)])[:, None]
    attr2d = jnp.concatenate([attributes, jnp.full((pad_n,), N_ATTRS, I32)])[:, None]

    zeros_pool = jnp.zeros((NROWS, 128), F32)
    zeros_cnt = jnp.zeros((NROWS, 16), F32)
    ones_cnt = jnp.ones((128, 16), F32)

    prm = params
    obj0 = _embed(objs2d, attr2d, prm['obj_emb'], prm['attr_emb'])
    pred = _pred_embed(pp.reshape(E2, 1), prm['pred_emb'])
    cnts = _sc_counts(sto, ones_cnt, zeros_cnt)

    obj_vecs = obj0
    for layer in prm['gconv']:
        (w1, b1), (w1b, b1b) = layer['net1']
        (v1, c1), (v2, c2) = layer['net2']
        cs, co = _sc_gather(obj_vecs, sto)
        ns4, newp, no4 = _edge_mlp(cs, pred, co,
                                   w1, b1[None, :], w1b, b1b[None, :])
        pooled4 = _sc_scatter(ns4, no4, sto, zeros_pool)
        obj_vecs = _node_mlp(pooled4, cnts, v1, c1[None, :], v2, c2[None, :])
        pred = newp

    (wb1, bb1), (wb2, bb2) = prm['box_net']
    (wa1, ba1), (wa2, ba2) = prm['angle_net']
    wb2p = jnp.pad(wb2, ((0, 0), (0, 128 - wb2.shape[1])))
    bb2p = jnp.pad(bb2, (0, 128 - bb2.shape[0]))
    wa2p = jnp.pad(wa2, ((0, 0), (0, 128 - wa2.shape[1])))
    ba2p = jnp.pad(ba2, (0, 128 - ba2.shape[0]),
                   constant_values=-1e30)
    boxes, angles = _heads(obj_vecs[:N], obj0[:N], z,
                           wb1, bb1[None, :], wb2p, bb2p[None, :],
                           wa1, ba1[None, :], wa2p, ba2p[None, :])
    return boxes[:, :6], angles[:, :24]


# speedup vs baseline: 2.7362x; 1.1641x over previous
import functools

import jax
import jax.numpy as jnp
from jax import lax
from jax.experimental import pallas as pl
from jax.experimental.pallas import tpu as pltpu
from jax.experimental.pallas import tpu_sc as plsc

F32 = jnp.float32
I32 = jnp.int32

N_OBJS = 36
N_PREDS = 16
N_ATTRS = 8
EMB = 128
H = 512
N = 10000
E = 160000
N2 = 10240
E2 = 163840

NC = 2
NS = 16
NW = NC * NS

EB = 1280
NB = 1280
GCH = E2 // NW
GIT = GCH // 128
SCH = E2 // NS
SIT = SCH // 128
SB = 32
SIT40 = SCH // SB
NSTG = 4
SHALF = SIT40 // NSTG
NROWS = N2 // NS

_mesh = plsc.VectorSubcoreMesh(core_axis_name="c", subcore_axis_name="s")


@functools.partial(
    pl.kernel,
    mesh=_mesh,
    out_type=[jax.ShapeDtypeStruct((E2, EMB), F32),
              jax.ShapeDtypeStruct((E2, EMB), F32)],
    scratch_types=[pltpu.VMEM((GIT, 128), I32),
                   pltpu.VMEM((GIT, 128), I32),
                   pltpu.VMEM((2, 128, EMB), F32),
                   pltpu.VMEM((2, 128, EMB), F32),
                   pltpu.SemaphoreType.DMA,
                   pltpu.SemaphoreType.DMA,
                   pltpu.SemaphoreType.DMA,
                   pltpu.SemaphoreType.DMA],
)
def _sc_gather(table_hbm, sto_hbm, cs_hbm, co_hbm,
               sidx_v, oidx_v, srows_v, orows_v, gs0, gs1, go0, go1):
    cid = lax.axis_index("c")
    sid = lax.axis_index("s")
    wid = sid * NC + cid
    pltpu.sync_copy(sto_hbm.at[0, pl.ds(wid * GIT, GIT)], sidx_v)
    pltpu.sync_copy(sto_hbm.at[1, pl.ds(wid * GIT, GIT)], oidx_v)

    pltpu.async_copy(table_hbm.at[sidx_v.at[0]], srows_v.at[0], gs0)
    pltpu.async_copy(table_hbm.at[oidx_v.at[0]], orows_v.at[0], go0)
    pltpu.async_copy(table_hbm.at[sidx_v.at[1]], srows_v.at[1], gs1)
    pltpu.async_copy(table_hbm.at[oidx_v.at[1]], orows_v.at[1], go1)

    def body(jj, carry):
        for b, gs, go in ((0, gs0, go0), (1, gs1, go1)):
            j = jj * 2 + b
            base = wid * GCH + j * 128
            pltpu.make_async_copy(
                table_hbm.at[sidx_v.at[j]], srows_v.at[b], gs).wait()
            pltpu.sync_copy(srows_v.at[b], cs_hbm.at[pl.ds(base, 128)])

            @pl.when(j + 2 < GIT)
            def _(j=j, b=b, gs=gs):
                pltpu.async_copy(
                    table_hbm.at[sidx_v.at[j + 2]], srows_v.at[b], gs)

            pltpu.make_async_copy(
                table_hbm.at[oidx_v.at[j]], orows_v.at[b], go).wait()
            pltpu.sync_copy(orows_v.at[b], co_hbm.at[pl.ds(base, 128)])

            @pl.when(j + 2 < GIT)
            def _(j=j, b=b, go=go):
                pltpu.async_copy(
                    table_hbm.at[oidx_v.at[j + 2]], orows_v.at[b], go)
        return carry

    lax.fori_loop(0, GIT // 2, body, 0)


@functools.partial(
    pl.kernel,
    mesh=_mesh,
    out_type=jax.ShapeDtypeStruct((4, N2, 128), F32),
    scratch_types=[pltpu.VMEM((SHALF, SB), I32),
                   pltpu.VMEM((SHALF, SB), I32),
                   pltpu.VMEM((2, SB, 128), F32),
                   pltpu.VMEM((2, SB, 128), F32),
                   pltpu.VMEM_SHARED((N2, 128), F32),
                   pltpu.SemaphoreType.DMA,
                   pltpu.SemaphoreType.DMA,
                   pltpu.SemaphoreType.DMA,
                   pltpu.SemaphoreType.DMA],
)
def _sc_scatter(news_hbm, newo_hbm, sto64_hbm, zeros_hbm, out_hbm,
                sidx_v, oidx_v, bs_v, bo_v, pool_sp, rs0, rs1, ro0, ro1):
    cid = lax.axis_index("c")
    sid = lax.axis_index("s")
    for chunk in range(2):
        c = cid * 2 + chunk
        pltpu.sync_copy(zeros_hbm, pool_sp.at[pl.ds(sid * NROWS, NROWS)])
        plsc.subcore_barrier()

        for h in range(NSTG):
            pltpu.sync_copy(
                sto64_hbm.at[0, pl.ds(sid * SIT40 + h * SHALF, SHALF)], sidx_v)
            pltpu.sync_copy(
                sto64_hbm.at[1, pl.ds(sid * SIT40 + h * SHALF, SHALF)], oidx_v)
            for b, rs, ro in ((0, rs0, ro0), (1, rs1, ro1)):
                base = sid * SCH + (h * SHALF + b) * SB
                pltpu.async_copy(news_hbm.at[c, pl.ds(base, SB)],
                                 bs_v.at[b], rs)
                pltpu.async_copy(newo_hbm.at[c, pl.ds(base, SB)],
                                 bo_v.at[b], ro)

            def body(ii, carry):
                for b, rs, ro in ((0, rs0, ro0), (1, rs1, ro1)):
                    i = ii * 2 + b
                    base = sid * SCH + (h * SHALF + i) * SB
                    pltpu.make_async_copy(
                        news_hbm.at[c, pl.ds(base, SB)], bs_v.at[b], rs).wait()
                    pltpu.sync_copy(bs_v.at[b], pool_sp.at[sidx_v.at[i]],
                                    add=True)

                    @pl.when(i + 2 < SHALF)
                    def _(base=base, b=b, rs=rs):
                        pltpu.async_copy(
                            news_hbm.at[c, pl.ds(base + 2 * SB, SB)],
                            bs_v.at[b], rs)

                    pltpu.make_async_copy(
                        newo_hbm.at[c, pl.ds(base, SB)], bo_v.at[b], ro).wait()
                    pltpu.sync_copy(bo_v.at[b], pool_sp.at[oidx_v.at[i]],
                                    add=True)

                    @pl.when(i + 2 < SHALF)
                    def _(base=base, b=b, ro=ro):
                        pltpu.async_copy(
                            newo_hbm.at[c, pl.ds(base + 2 * SB, SB)],
                            bo_v.at[b], ro)
                return carry

            lax.fori_loop(0, SHALF // 2, body, 0)
        plsc.subcore_barrier()
        pltpu.sync_copy(pool_sp.at[pl.ds(sid * NROWS, NROWS)],
                        out_hbm.at[c, pl.ds(sid * NROWS, NROWS)])
        plsc.subcore_barrier()


@functools.partial(
    pl.kernel,
    mesh=_mesh,
    out_type=jax.ShapeDtypeStruct((2, N2, 16), F32),
    scratch_types=[pltpu.VMEM((SIT, 128), I32),
                   pltpu.VMEM((128, 16), F32),
                   pltpu.VMEM_SHARED((N2, 16), F32)],
)
def _sc_counts(sto_hbm, ones_hbm, zeros_hbm, out_hbm, idx_v, ones_v, cnt_sp):
    cid = lax.axis_index("c")
    sid = lax.axis_index("s")
    pltpu.sync_copy(sto_hbm.at[cid, pl.ds(sid * SIT, SIT)], idx_v)
    pltpu.sync_copy(ones_hbm, ones_v)
    pltpu.sync_copy(zeros_hbm, cnt_sp.at[pl.ds(sid * NROWS, NROWS)])
    plsc.subcore_barrier()

    def body(j, carry):
        pltpu.sync_copy(ones_v, cnt_sp.at[idx_v.at[j]], add=True)
        return carry

    lax.fori_loop(0, SIT, body, 0)
    plsc.subcore_barrier()
    pltpu.sync_copy(cnt_sp.at[pl.ds(sid * NROWS, NROWS)],
                    out_hbm.at[cid, pl.ds(sid * NROWS, NROWS)])


def _embed_body(objs_ref, attr_ref, oemb_ref, aemb_ref, out_ref):
    oids = objs_ref[...]
    aids = attr_ref[...]
    oh_o = (oids == lax.broadcasted_iota(I32, (NB, N_OBJS), 1)).astype(F32)
    oh_a = (aids == lax.broadcasted_iota(I32, (NB, N_ATTRS), 1)).astype(F32)
    ov = jnp.dot(oh_o, oemb_ref[...], preferred_element_type=F32)
    av = jnp.dot(oh_a, aemb_ref[...], preferred_element_type=F32)
    out_ref[...] = jnp.concatenate([ov, av], axis=1)


def _embed(objs2d, attr2d, oemb, aemb):
    return pl.pallas_call(
        _embed_body,
        grid=(N2 // NB,),
        in_specs=[pl.BlockSpec((NB, 1), lambda i: (i, 0)),
                  pl.BlockSpec((NB, 1), lambda i: (i, 0)),
                  pl.BlockSpec(oemb.shape, lambda i: (0, 0)),
                  pl.BlockSpec(aemb.shape, lambda i: (0, 0))],
        out_specs=pl.BlockSpec((NB, EMB), lambda i: (i, 0)),
        out_shape=jax.ShapeDtypeStruct((N2, EMB), F32),
    )(objs2d, attr2d, oemb, aemb)


def _pred_body(p_ref, pemb_ref, out_ref):
    pids = p_ref[...]
    oh = (pids == lax.broadcasted_iota(I32, (EB, N_PREDS), 1)).astype(F32)
    out_ref[...] = jnp.dot(oh, pemb_ref[...], preferred_element_type=F32)


def _pred_embed(p2d, pemb):
    return pl.pallas_call(
        _pred_body,
        grid=(E2 // EB,),
        in_specs=[pl.BlockSpec((EB, 1), lambda i: (i, 0)),
                  pl.BlockSpec(pemb.shape, lambda i: (0, 0))],
        out_specs=pl.BlockSpec((EB, EMB), lambda i: (i, 0)),
        out_shape=jax.ShapeDtypeStruct((E2, EMB), F32),
    )(p2d, pemb)


BF16 = jnp.bfloat16


def _edge_body(cs_ref, pv_ref, co_ref, w1_ref, b1_ref, w2_ref, b2_ref,
               ns_ref, np_ref, no_ref):
    csb = cs_ref[...].astype(BF16)
    pvb = pv_ref[...].astype(BF16)
    cob = co_ref[...].astype(BF16)
    h = (jnp.dot(csb, w1_ref[0:EMB], preferred_element_type=F32)
         + jnp.dot(pvb, w1_ref[EMB:2 * EMB], preferred_element_type=F32)
         + jnp.dot(cob, w1_ref[2 * EMB:3 * EMB], preferred_element_type=F32)
         + b1_ref[...])
    h = jnp.maximum(h, 0.0).astype(BF16)
    for c in range(4):
        ns_ref[c] = jnp.maximum(
            jnp.dot(h, w2_ref[:, c * 128:(c + 1) * 128],
                    preferred_element_type=F32) + b2_ref[:, c * 128:(c + 1) * 128],
            0.0)
    np_ref[...] = jnp.maximum(
        jnp.dot(h, w2_ref[:, 512:640], preferred_element_type=F32)
        + b2_ref[:, 512:640], 0.0)
    for c in range(4):
        lo = 640 + c * 128
        no_ref[c] = jnp.maximum(
            jnp.dot(h, w2_ref[:, lo:lo + 128], preferred_element_type=F32)
            + b2_ref[:, lo:lo + 128], 0.0)


def _edge_mlp(cs, pv, co, w1, b1, w2, b2):
    return pl.pallas_call(
        _edge_body,
        grid=(E2 // EB,),
        in_specs=[pl.BlockSpec((EB, EMB), lambda i: (i, 0)),
                  pl.BlockSpec((EB, EMB), lambda i: (i, 0)),
                  pl.BlockSpec((EB, EMB), lambda i: (i, 0)),
                  pl.BlockSpec(w1.shape, lambda i: (0, 0)),
                  pl.BlockSpec(b1.shape, lambda i: (0, 0)),
                  pl.BlockSpec(w2.shape, lambda i: (0, 0)),
                  pl.BlockSpec(b2.shape, lambda i: (0, 0))],
        out_specs=[pl.BlockSpec((4, EB, 128), lambda i: (0, i, 0)),
                   pl.BlockSpec((EB, EMB), lambda i: (i, 0)),
                   pl.BlockSpec((4, EB, 128), lambda i: (0, i, 0))],
        out_shape=[jax.ShapeDtypeStruct((4, E2, 128), F32),
                   jax.ShapeDtypeStruct((E2, EMB), F32),
                   jax.ShapeDtypeStruct((4, E2, 128), F32)],
    )(cs, pv, co, w1, b1, w2, b2)


def _node_body(p4_ref, cnt_ref, w1_ref, b1_ref, w2_ref, b2_ref, out_ref):
    cnt = cnt_ref[0][:, 0:1] + cnt_ref[1][:, 0:1]
    inv = 1.0 / jnp.maximum(cnt, 1.0)
    x = jnp.concatenate([p4_ref[0], p4_ref[1], p4_ref[2], p4_ref[3]], axis=1)
    x = x * inv
    h = jnp.maximum(jnp.dot(x, w1_ref[...], preferred_element_type=F32)
                    + b1_ref[...], 0.0)
    out_ref[...] = jnp.maximum(
        jnp.dot(h, w2_ref[...], preferred_element_type=F32) + b2_ref[...], 0.0)


def _node_mlp(p4, cnts, w1, b1, w2, b2):
    return pl.pallas_call(
        _node_body,
        grid=(N2 // NB,),
        in_specs=[pl.BlockSpec((4, NB, 128), lambda i: (0, i, 0)),
                  pl.BlockSpec((2, NB, 16), lambda i: (0, i, 0)),
                  pl.BlockSpec(w1.shape, lambda i: (0, 0)),
                  pl.BlockSpec(b1.shape, lambda i: (0, 0)),
                  pl.BlockSpec(w2.shape, lambda i: (0, 0)),
                  pl.BlockSpec(b2.shape, lambda i: (0, 0))],
        out_specs=pl.BlockSpec((NB, EMB), lambda i: (i, 0)),
        out_shape=jax.ShapeDtypeStruct((N2, EMB), F32),
    )(p4, cnts, w1, b1, w2, b2)


HB = 1000


def _heads_body(obj_ref, obj0_ref, z_ref,
                wb1_ref, bb1_ref, wb2_ref, bb2_ref,
                wa1_ref, ba1_ref, wa2_ref, ba2_ref,
                box_ref, ang_ref):
    obj = obj_ref[...]
    zz = z_ref[...]
    attr = obj0_ref[:, 96:128]
    bx = jnp.concatenate([obj, zz, attr], axis=1)
    hb = jnp.maximum(jnp.dot(bx, wb1_ref[...], preferred_element_type=F32)
                     + bb1_ref[...], 0.0)
    box_ref[...] = jnp.dot(hb, wb2_ref[...], preferred_element_type=F32) + bb2_ref[...]
    ax = jnp.concatenate([obj, zz], axis=1)
    ha = jnp.maximum(jnp.dot(ax, wa1_ref[...], preferred_element_type=F32)
                     + ba1_ref[...], 0.0)
    a = jnp.dot(ha, wa2_ref[...], preferred_element_type=F32) + ba2_ref[...]
    m = jnp.max(a, axis=1, keepdims=True)
    lse = m + jnp.log(jnp.sum(jnp.exp(a - m), axis=1, keepdims=True))
    ang_ref[...] = a - lse


def _heads(obj, obj0, z, wb1, bb1, wb2, bb2, wa1, ba1, wa2, ba2):
    return pl.pallas_call(
        _heads_body,
        grid=(N // HB,),
        in_specs=[pl.BlockSpec((HB, EMB), lambda i: (i, 0)),
                  pl.BlockSpec((HB, EMB), lambda i: (i, 0)),
                  pl.BlockSpec((HB, EMB), lambda i: (i, 0)),
                  pl.BlockSpec(wb1.shape, lambda i: (0, 0)),
                  pl.BlockSpec(bb1.shape, lambda i: (0, 0)),
                  pl.BlockSpec(wb2.shape, lambda i: (0, 0)),
                  pl.BlockSpec(bb2.shape, lambda i: (0, 0)),
                  pl.BlockSpec(wa1.shape, lambda i: (0, 0)),
                  pl.BlockSpec(ba1.shape, lambda i: (0, 0)),
                  pl.BlockSpec(wa2.shape, lambda i: (0, 0)),
                  pl.BlockSpec(ba2.shape, lambda i: (0, 0))],
        out_specs=[pl.BlockSpec((HB, 128), lambda i: (i, 0)),
                   pl.BlockSpec((HB, 128), lambda i: (i, 0))],
        out_shape=[jax.ShapeDtypeStruct((N, 128), F32),
                   jax.ShapeDtypeStruct((N, 128), F32)],
    )(obj, obj0, z, wb1, bb1, wb2, bb2, wa1, ba1, wa2, ba2)


def kernel(z, objs, triples, attributes, params):
    s = triples[:, 0]
    p = triples[:, 1]
    o = triples[:, 2]
    pad_e = E2 - E
    sp = jnp.concatenate([s, jnp.full((pad_e,), N, I32)])
    op = jnp.concatenate([o, jnp.full((pad_e,), N, I32)])
    pp = jnp.concatenate([p, jnp.zeros((pad_e,), I32)])
    sto = jnp.stack([sp.reshape(E2 // 128, 128), op.reshape(E2 // 128, 128)])
    sto64 = jnp.stack([sp.reshape(E2 // SB, SB), op.reshape(E2 // SB, SB)])

    pad_n = N2 - N
    objs2d = jnp.concatenate([objs, jnp.full((pad_n,), N_OBJS, I32)])[:, None]
    attr2d = jnp.concatenate([attributes, jnp.full((pad_n,), N_ATTRS, I32)])[:, None]

    zeros_pool = jnp.zeros((NROWS, 128), F32)
    zeros_cnt = jnp.zeros((NROWS, 16), F32)
    ones_cnt = jnp.ones((128, 16), F32)

    prm = params
    obj0 = _embed(objs2d, attr2d, prm['obj_emb'], prm['attr_emb'])
    pred = _pred_embed(pp.reshape(E2, 1), prm['pred_emb'])
    cnts = _sc_counts(sto, ones_cnt, zeros_cnt)

    obj_vecs = obj0
    for layer in prm['gconv']:
        (w1, b1), (w1b, b1b) = layer['net1']
        (v1, c1), (v2, c2) = layer['net2']
        cs, co = _sc_gather(obj_vecs, sto)
        ns4, newp, no4 = _edge_mlp(cs, pred, co,
                                   w1.astype(BF16), b1[None, :],
                                   w1b.astype(BF16), b1b[None, :])
        pooled4 = _sc_scatter(ns4, no4, sto64, zeros_pool)
        obj_vecs = _node_mlp(pooled4, cnts, v1, c1[None, :], v2, c2[None, :])
        pred = newp

    (wb1, bb1), (wb2, bb2) = prm['box_net']
    (wa1, ba1), (wa2, ba2) = prm['angle_net']
    wb2p = jnp.pad(wb2, ((0, 0), (0, 128 - wb2.shape[1])))
    bb2p = jnp.pad(bb2, (0, 128 - bb2.shape[0]))
    wa2p = jnp.pad(wa2, ((0, 0), (0, 128 - wa2.shape[1])))
    ba2p = jnp.pad(ba2, (0, 128 - ba2.shape[0]),
                   constant_values=-1e30)
    boxes, angles = _heads(obj_vecs[:N], obj0[:N], z,
                           wb1, bb1[None, :], wb2p, bb2p[None, :],
                           wa1, ba1[None, :], wa2p, ba2p[None, :])
    return boxes[:, :6], angles[:, :24]
